# Initial kernel scaffold; baseline (speedup 1.0000x reference)
import functools

import jax
import jax.numpy as jnp
from jax import lax
from jax.experimental import pallas as pl
from jax.experimental.pallas import tpu as pltpu
from jax.experimental.pallas import tpu_sc as plsc

EMB = 100
HID = 200
G = 256
NTASK = 128
NLAYER = 5
N = 50000
E = 800000
D = 112
CH = 224
NCH = 224
NPAD = CH * NCH
KE = 128
NW = 32
RPW = NCH // NW
TOT = E + NCH * KE
NB = 2000
NSTEP = N // NB


def _pad_d(x):
  return jnp.pad(x, [(0, 0)] * (x.ndim - 1) + [(0, D - x.shape[-1])])


def _sc_body(hin, etab, srcp, eidxp, ldstp, startsw, nitw, out,
             etab_v, aggr_v, src_v, eidx_v, ldst_v, hbuf, st_v, ni_v, sem):
  w = lax.axis_index("s") * 2 + lax.axis_index("c")
  pltpu.sync_copy(etab, etab_v)
  pltpu.sync_copy(startsw.at[w], st_v)
  pltpu.sync_copy(nitw.at[w], ni_v)
  st_vec = st_v[...]
  ni_vec = ni_v[...]
  zero16 = jnp.zeros((16,), jnp.float32)

  for cc in range(RPW):
    chunk = w * RPW + cc
    s0 = st_vec[cc]
    n0 = ni_vec[cc]

    def zrow(r, _):
      for j in range(D // 16):
        aggr_v[r, pl.ds(16 * j, 16)] = zero16
      return 0
    lax.fori_loop(0, CH + 1, zrow, 0)

    def etile(t, _):
      off = s0 + t * KE
      pltpu.sync_copy(srcp.at[pl.ds(off, KE)], src_v)
      pltpu.sync_copy(eidxp.at[pl.ds(off, KE)], eidx_v)
      pltpu.sync_copy(ldstp.at[pl.ds(off, KE)], ldst_v)
      pltpu.async_copy(hin.at[src_v], hbuf, sem).wait()

      def grp(g, _):
        lvec = ldst_v[pl.ds(g * 16, 16)]
        evec = eidx_v[pl.ds(g * 16, 16)]
        for k in range(16):
          li = lvec[k]
          ei = evec[k]
          row = g * 16 + k
          for j in range(D // 16):
            m = jnp.maximum(
                hbuf[row, pl.ds(16 * j, 16)] + etab_v[ei, pl.ds(16 * j, 16)],
                0.0)
            plsc.addupdate(aggr_v.at[li, pl.ds(16 * j, 16)], m)
        return 0
      lax.fori_loop(0, KE // 16, grp, 0)
      return 0
    lax.fori_loop(0, n0, etile, 0)
    pltpu.sync_copy(aggr_v.at[pl.ds(0, CH)], out.at[pl.ds(chunk * CH, CH)])


_sc_aggr = functools.partial(
    pl.kernel,
    out_type=jax.ShapeDtypeStruct((NPAD, D), jnp.float32),
    mesh=plsc.VectorSubcoreMesh(core_axis_name="c", subcore_axis_name="s"),
    scratch_types=[
        pltpu.VMEM((512, D), jnp.float32),
        pltpu.VMEM((CH + 1, D), jnp.float32),
        pltpu.VMEM((KE,), jnp.int32),
        pltpu.VMEM((KE,), jnp.int32),
        pltpu.VMEM((KE,), jnp.int32),
        pltpu.VMEM((KE, D), jnp.float32),
        pltpu.VMEM((16,), jnp.int32),
        pltpu.VMEM((16,), jnp.int32),
        pltpu.SemaphoreType.DMA,
    ],
)(_sc_body)


def _atom_body(x_ref, af_ref, h_ref):
  x = x_ref[...]
  iota = lax.broadcasted_iota(jnp.int32, (NB, 576), 1)
  oh = jnp.zeros((NB, 576), jnp.float32)
  for i in range(9):
    oh += (iota == (x[:, i:i + 1] + 64 * i)).astype(jnp.float32)
  h_ref[...] = jnp.dot(oh, af_ref[...], preferred_element_type=jnp.float32)


def _atom_call(x_feat, atom_flat):
  return pl.pallas_call(
      _atom_body,
      grid=(NSTEP,),
      in_specs=[
          pl.BlockSpec((NB, 9), lambda i: (i, 0)),
          pl.BlockSpec((576, D), lambda i: (0, 0)),
      ],
      out_specs=pl.BlockSpec((NB, D), lambda i: (i, 0)),
      out_shape=jax.ShapeDtypeStruct((N, D), jnp.float32),
  )(x_feat, atom_flat)


def _a_body(h_ref, b_ref, vn_ref, hin_ref, ss_ref):
  i = pl.program_id(0)
  ohT = (lax.broadcasted_iota(jnp.int32, (G, NB), 0) == b_ref[0]).astype(
      jnp.float32)
  vnb = lax.dot_general(ohT, vn_ref[...], (((0,), (0,)), ((), ())),
                        preferred_element_type=jnp.float32)
  hin = h_ref[...] + vnb
  hin_ref[...] = hin
  part = lax.dot_general(ohT, hin, (((1,), (0,)), ((), ())),
                         preferred_element_type=jnp.float32)

  @pl.when(i == 0)
  def _():
    ss_ref[...] = jnp.zeros_like(ss_ref)
  ss_ref[...] += part


def _a_call(h, batch3, vn):
  return pl.pallas_call(
      _a_body,
      grid=(NSTEP,),
      in_specs=[
          pl.BlockSpec((NB, D), lambda i: (i, 0)),
          pl.BlockSpec((1, 1, NB), lambda i: (i, 0, 0)),
          pl.BlockSpec((G, D), lambda i: (0, 0)),
      ],
      out_specs=[
          pl.BlockSpec((NB, D), lambda i: (i, 0)),
          pl.BlockSpec((G, D), lambda i: (0, 0)),
      ],
      out_shape=[
          jax.ShapeDtypeStruct((N, D), jnp.float32),
          jax.ShapeDtypeStruct((G, D), jnp.float32),
      ],
  )(h, batch3, vn)


def _b_body(relu_out, hin_ref, ag_ref, eps_ref, w1_ref, b1_ref, g1_ref,
            be1_ref, w2_ref, b2_ref, bg_ref, bb_ref, out_ref):
  z = (1.0 + eps_ref[0, 0]) * hin_ref[...] + ag_ref[...]
  z = jnp.dot(z, w1_ref[...], preferred_element_type=jnp.float32) + b1_ref[...]
  z = z * g1_ref[...] + be1_ref[...]
  z = jnp.maximum(z, 0.0)
  z = jnp.dot(z, w2_ref[...], preferred_element_type=jnp.float32) + b2_ref[...]
  z = z * bg_ref[...] + bb_ref[...]
  if relu_out:
    z = jnp.maximum(z, 0.0)
  out_ref[...] = z


def _b_call(relu_out, hin, aggr, eps, w1, b1, g1, be1, w2, b2, bg, bb):
  row = lambda i: (0, 0)
  return pl.pallas_call(
      functools.partial(_b_body, relu_out),
      grid=(NSTEP,),
      in_specs=[
          pl.BlockSpec((NB, D), lambda i: (i, 0)),
          pl.BlockSpec((NB, D), lambda i: (i, 0)),
          pl.BlockSpec((1, 1), row),
          pl.BlockSpec((D, HID), row),
          pl.BlockSpec((1, HID), row),
          pl.BlockSpec((1, HID), row),
          pl.BlockSpec((1, HID), row),
          pl.BlockSpec((HID, D), row),
          pl.BlockSpec((1, D), row),
          pl.BlockSpec((1, D), row),
          pl.BlockSpec((1, D), row),
      ],
      out_specs=pl.BlockSpec((NB, D), lambda i: (i, 0)),
      out_shape=jax.ShapeDtypeStruct((N, D), jnp.float32),
  )(hin, aggr, eps, w1, b1, g1, be1, w2, b2, bg, bb)


def _v_body(ss_ref, vn_ref, w1_ref, b1_ref, g1_ref, be1_ref, w2_ref, b2_ref,
            g2_ref, be2_ref, out_ref):
  vt = ss_ref[...] + vn_ref[...]
  v = jnp.dot(vt, w1_ref[...], preferred_element_type=jnp.float32) + b1_ref[...]
  v = jnp.maximum(v * g1_ref[...] + be1_ref[...], 0.0)
  v = jnp.dot(v, w2_ref[...], preferred_element_type=jnp.float32) + b2_ref[...]
  v = v * g2_ref[...] + be2_ref[...]
  out_ref[...] = jnp.maximum(v, 0.0)


def _v_call(ss, vn, w1, b1, g1, be1, w2, b2, g2, be2):
  return pl.pallas_call(
      _v_body,
      out_shape=jax.ShapeDtypeStruct((G, D), jnp.float32),
  )(ss, vn, w1, b1, g1, be1, w2, b2, g2, be2)


def _pool_body(h_ref, b_ref, ss_ref):
  i = pl.program_id(0)
  ohT = (lax.broadcasted_iota(jnp.int32, (G, NB), 0) == b_ref[0]).astype(
      jnp.float32)
  part = lax.dot_general(ohT, h_ref[...], (((1,), (0,)), ((), ())),
                         preferred_element_type=jnp.float32)

  @pl.when(i == 0)
  def _():
    ss_ref[...] = jnp.zeros_like(ss_ref)
  ss_ref[...] += part


def _pool_call(h, batch3):
  return pl.pallas_call(
      _pool_body,
      grid=(NSTEP,),
      in_specs=[
          pl.BlockSpec((NB, D), lambda i: (i, 0)),
          pl.BlockSpec((1, 1, NB), lambda i: (i, 0, 0)),
      ],
      out_specs=pl.BlockSpec((G, D), lambda i: (0, 0)),
      out_shape=jax.ShapeDtypeStruct((G, D), jnp.float32),
  )(h, batch3)


def _pred_body(hg_ref, fp_ref, w1h_ref, w1f_ref, b1_ref, w2_ref, b2_ref,
               out_ref):
  z = (jnp.dot(hg_ref[...], w1h_ref[...], preferred_element_type=jnp.float32)
       + jnp.dot(fp_ref[...], w1f_ref[...], preferred_element_type=jnp.float32)
       + b1_ref[...])
  z = jax.nn.gelu(z)
  out_ref[...] = (
      jnp.dot(z, w2_ref[...], preferred_element_type=jnp.float32) + b2_ref[...])


def _pred_call(hg, fp, w1h, w1f, b1, w2, b2):
  return pl.pallas_call(
      _pred_body,
      out_shape=jax.ShapeDtypeStruct((G, NTASK), jnp.float32),
  )(hg, fp, w1h, w1f, b1, w2, b2)


def _etab_body(bf_ref, et_ref):
  ci = lax.broadcasted_iota(jnp.int32, (512, 24), 0)
  cj = lax.broadcasted_iota(jnp.int32, (512, 24), 1)
  oh = ((cj == (ci >> 6)) | (cj == 8 + ((ci >> 3) & 7)) |
        (cj == 16 + (ci & 7))).astype(jnp.float32)
  et_ref[0] = jnp.dot(oh, bf_ref[0], preferred_element_type=jnp.float32)


def _etab_call(bond_flat):
  return pl.pallas_call(
      _etab_body,
      grid=(NLAYER,),
      in_specs=[pl.BlockSpec((1, 24, D), lambda l: (l, 0, 0))],
      out_specs=pl.BlockSpec((1, 512, D), lambda l: (l, 0, 0)),
      out_shape=jax.ShapeDtypeStruct((NLAYER, 512, D), jnp.float32),
  )(bond_flat)


def kernel(x_feat, edge_index, edge_attr, batch, fp, atom_emb, vn_emb0,
           bond_emb, conv_eps, conv_W1, conv_b1, conv_g1, conv_be1, conv_W2,
           conv_b2, bn_g, bn_b, vn_W1, vn_b1, vn_g1, vn_be1, vn_W2, vn_b2,
           vn_g2, vn_be2, pred_W1, pred_b1, pred_W2, pred_b2):
  src = edge_index[0]
  dst = edge_index[1]

  c = dst // CH
  order = jnp.argsort(c)
  cs = c[order]
  counts = jnp.bincount(c, length=NCH)
  cnt_pad = ((counts + KE - 1) // KE) * KE
  starts_pad = jnp.concatenate(
      [jnp.zeros((1,), cnt_pad.dtype), jnp.cumsum(cnt_pad)[:-1]])
  starts_orig = jnp.cumsum(counts) - counts
  rank = jnp.arange(E, dtype=cs.dtype) - starts_orig[cs]
  pos = starts_pad[cs] + rank
  eidx = edge_attr[:, 0] * 64 + edge_attr[:, 1] * 8 + edge_attr[:, 2]
  src_p = jnp.zeros((TOT,), jnp.int32).at[pos].set(src[order])
  eidx_p = jnp.zeros((TOT,), jnp.int32).at[pos].set(eidx[order])
  ldst_p = jnp.full((TOT,), CH, jnp.int32).at[pos].set((dst - c * CH)[order])
  starts_w = jnp.zeros((NW, 16), jnp.int32).at[:, :RPW].set(
      starts_pad.reshape(NW, RPW).astype(jnp.int32))
  nit_w = jnp.zeros((NW, 16), jnp.int32).at[:, :RPW].set(
      (cnt_pad // KE).reshape(NW, RPW).astype(jnp.int32))

  atom_flat = _pad_d(atom_emb.reshape(576, EMB))
  bond_flat = _pad_d(bond_emb.reshape(NLAYER, 24, EMB))
  w1p = jnp.pad(conv_W1, ((0, 0), (0, D - EMB), (0, 0)))
  w2p = _pad_d(conv_W2)
  b2p = _pad_d(conv_b2)
  bgp = _pad_d(bn_g)
  bbp = _pad_d(bn_b)
  vw1p = jnp.pad(vn_W1, ((0, 0), (0, D - EMB), (0, 0)))
  vw2p = _pad_d(vn_W2)
  vb2p = _pad_d(vn_b2)
  vg2p = _pad_d(vn_g2)
  vbe2p = _pad_d(vn_be2)
  w1h = jnp.pad(pred_W1[:EMB], ((0, D - EMB), (0, 0)))
  w1f = pred_W1[EMB:]
  batch3 = batch.reshape(NSTEP, 1, NB)
  r1 = lambda x: x.reshape(1, -1)

  et = _etab_call(bond_flat)
  h = _atom_call(x_feat, atom_flat)
  vn = jnp.broadcast_to(_pad_d(vn_emb0), (G, D))

  for l in range(NLAYER):
    hin, ss = _a_call(h, batch3, vn)
    aggr = _sc_aggr(hin, et[l], src_p, eidx_p, ldst_p, starts_w, nit_w)[:N]
    h = _b_call(l < NLAYER - 1, hin, aggr, conv_eps[l].reshape(1, 1),
                w1p[l], r1(conv_b1[l]), r1(conv_g1[l]), r1(conv_be1[l]),
                w2p[l], r1(b2p[l]), r1(bgp[l]), r1(bbp[l]))
    if l < NLAYER - 1:
      vn = _v_call(ss, vn, vw1p[l], r1(vn_b1[l]), r1(vn_g1[l]),
                   r1(vn_be1[l]), vw2p[l], r1(vb2p[l]), r1(vg2p[l]),
                   r1(vbe2p[l]))

  hg = _pool_call(h, batch3)
  return _pred_call(hg, fp, w1h, w1f, r1(pred_b1), pred_W2, r1(pred_b2))


# speedup vs baseline: 1.4473x; 1.4473x over previous
import functools

import jax
import jax.numpy as jnp
from jax import lax
from jax.experimental import pallas as pl
from jax.experimental.pallas import tpu as pltpu
from jax.experimental.pallas import tpu_sc as plsc

EMB = 100
HID = 200
G = 256
NTASK = 128
NLAYER = 5
N = 50000
E = 800000
D = 128
CH = 224
NCH = 224
NPAD = CH * NCH
KE = 128
NW = 32
RPW = NCH // NW
TOT = E + NCH * KE
NB = 2000
NSTEP = N // NB


def _pad_d(x):
  return jnp.pad(x, [(0, 0)] * (x.ndim - 1) + [(0, D - x.shape[-1])])


def _sc_body(hin, etab, srcp, eidxp, ldstp, startsw, nitw, out,
             etab_v, aggr_v, src_v, eidx_v, ldst_v, hbuf, st_v, ni_v, sem):
  w = lax.axis_index("s") * 2 + lax.axis_index("c")
  pltpu.sync_copy(etab, etab_v)
  pltpu.sync_copy(startsw.at[w], st_v)
  pltpu.sync_copy(nitw.at[w], ni_v)
  st_vec = st_v[...]
  ni_vec = ni_v[...]
  zero16 = jnp.zeros((16,), jnp.float32)

  for cc in range(RPW):
    chunk = w * RPW + cc
    s0 = st_vec[cc]
    n0 = ni_vec[cc]

    def zrow(r, _):
      for j in range(D // 16):
        aggr_v[r, pl.ds(16 * j, 16)] = zero16
      return 0
    lax.fori_loop(0, CH + 1, zrow, 0)

    def etile(t, _):
      off = pl.multiple_of(s0 + t * KE, KE)
      pltpu.sync_copy(srcp.at[pl.ds(off, KE)], src_v)
      pltpu.sync_copy(eidxp.at[pl.ds(off, KE)], eidx_v)
      pltpu.sync_copy(ldstp.at[pl.ds(off, KE)], ldst_v)
      pltpu.async_copy(hin.at[src_v], hbuf, sem).wait()

      def grp(g, _):
        lvec = ldst_v[pl.ds(g * 16, 16)]
        evec = eidx_v[pl.ds(g * 16, 16)]
        for k in range(16):
          li = lvec[k]
          ei = evec[k]
          row = g * 16 + k
          for j in range(D // 16):
            m = jnp.maximum(
                hbuf[row, pl.ds(16 * j, 16)] + etab_v[ei, pl.ds(16 * j, 16)],
                0.0)
            plsc.addupdate(aggr_v.at[li, pl.ds(16 * j, 16)], m)
        return 0
      lax.fori_loop(0, KE // 16, grp, 0)
      return 0
    lax.fori_loop(0, n0, etile, 0)
    pltpu.sync_copy(aggr_v.at[pl.ds(0, CH)], out.at[pl.ds(chunk * CH, CH)])


_sc_aggr = functools.partial(
    pl.kernel,
    out_type=jax.ShapeDtypeStruct((NPAD, D), jnp.float32),
    mesh=plsc.VectorSubcoreMesh(core_axis_name="c", subcore_axis_name="s",
                                num_cores=2, num_subcores=16),
    scratch_types=[
        pltpu.VMEM((512, D), jnp.float32),
        pltpu.VMEM((CH + 1, D), jnp.float32),
        pltpu.VMEM((KE,), jnp.int32),
        pltpu.VMEM((KE,), jnp.int32),
        pltpu.VMEM((KE,), jnp.int32),
        pltpu.VMEM((KE, D), jnp.float32),
        pltpu.VMEM((16,), jnp.int32),
        pltpu.VMEM((16,), jnp.int32),
        pltpu.SemaphoreType.DMA,
    ],
)(_sc_body)


def _atom_body(x_ref, af_ref, h_ref):
  x = x_ref[...]
  iota = lax.broadcasted_iota(jnp.int32, (NB, 576), 1)
  oh = jnp.zeros((NB, 576), jnp.float32)
  for i in range(9):
    oh += (iota == (x[:, i:i + 1] + 64 * i)).astype(jnp.float32)
  h_ref[...] = jnp.dot(oh, af_ref[...], preferred_element_type=jnp.float32)


def _atom_call(x_feat, atom_flat):
  return pl.pallas_call(
      _atom_body,
      grid=(NSTEP,),
      in_specs=[
          pl.BlockSpec((NB, 9), lambda i: (i, 0)),
          pl.BlockSpec((576, D), lambda i: (0, 0)),
      ],
      out_specs=pl.BlockSpec((NB, D), lambda i: (i, 0)),
      out_shape=jax.ShapeDtypeStruct((N, D), jnp.float32),
  )(x_feat, atom_flat)


def _a_body(h_ref, b_ref, vn_ref, hin_ref, ss_ref):
  i = pl.program_id(0)
  ohT = (lax.broadcasted_iota(jnp.int32, (G, NB), 0) == b_ref[0]).astype(
      jnp.float32)
  vnb = lax.dot_general(ohT, vn_ref[...], (((0,), (0,)), ((), ())),
                        preferred_element_type=jnp.float32)
  hin = h_ref[...] + vnb
  hin_ref[...] = hin
  part = lax.dot_general(ohT, hin, (((1,), (0,)), ((), ())),
                         preferred_element_type=jnp.float32)

  @pl.when(i == 0)
  def _():
    ss_ref[...] = jnp.zeros_like(ss_ref)
  ss_ref[...] += part


def _a_call(h, batch3, vn):
  return pl.pallas_call(
      _a_body,
      grid=(NSTEP,),
      in_specs=[
          pl.BlockSpec((NB, D), lambda i: (i, 0)),
          pl.BlockSpec((1, 1, NB), lambda i: (i, 0, 0)),
          pl.BlockSpec((G, D), lambda i: (0, 0)),
      ],
      out_specs=[
          pl.BlockSpec((NB, D), lambda i: (i, 0)),
          pl.BlockSpec((G, D), lambda i: (0, 0)),
      ],
      out_shape=[
          jax.ShapeDtypeStruct((N, D), jnp.float32),
          jax.ShapeDtypeStruct((G, D), jnp.float32),
      ],
  )(h, batch3, vn)


def _b_body(relu_out, hin_ref, ag_ref, eps_ref, w1_ref, b1_ref, g1_ref,
            be1_ref, w2_ref, b2_ref, bg_ref, bb_ref, out_ref):
  z = (1.0 + eps_ref[0, 0]) * hin_ref[...] + ag_ref[...]
  z = jnp.dot(z, w1_ref[...], preferred_element_type=jnp.float32) + b1_ref[...]
  z = z * g1_ref[...] + be1_ref[...]
  z = jnp.maximum(z, 0.0)
  z = jnp.dot(z, w2_ref[...], preferred_element_type=jnp.float32) + b2_ref[...]
  z = z * bg_ref[...] + bb_ref[...]
  if relu_out:
    z = jnp.maximum(z, 0.0)
  out_ref[...] = z


def _b_call(relu_out, hin, aggr, eps, w1, b1, g1, be1, w2, b2, bg, bb):
  row = lambda i: (0, 0)
  return pl.pallas_call(
      functools.partial(_b_body, relu_out),
      grid=(NSTEP,),
      in_specs=[
          pl.BlockSpec((NB, D), lambda i: (i, 0)),
          pl.BlockSpec((NB, D), lambda i: (i, 0)),
          pl.BlockSpec((1, 1), row),
          pl.BlockSpec((D, HID), row),
          pl.BlockSpec((1, HID), row),
          pl.BlockSpec((1, HID), row),
          pl.BlockSpec((1, HID), row),
          pl.BlockSpec((HID, D), row),
          pl.BlockSpec((1, D), row),
          pl.BlockSpec((1, D), row),
          pl.BlockSpec((1, D), row),
      ],
      out_specs=pl.BlockSpec((NB, D), lambda i: (i, 0)),
      out_shape=jax.ShapeDtypeStruct((N, D), jnp.float32),
  )(hin, aggr, eps, w1, b1, g1, be1, w2, b2, bg, bb)


def _v_body(ss_ref, vn_ref, w1_ref, b1_ref, g1_ref, be1_ref, w2_ref, b2_ref,
            g2_ref, be2_ref, out_ref):
  vt = ss_ref[...] + vn_ref[...]
  v = jnp.dot(vt, w1_ref[...], preferred_element_type=jnp.float32) + b1_ref[...]
  v = jnp.maximum(v * g1_ref[...] + be1_ref[...], 0.0)
  v = jnp.dot(v, w2_ref[...], preferred_element_type=jnp.float32) + b2_ref[...]
  v = v * g2_ref[...] + be2_ref[...]
  out_ref[...] = jnp.maximum(v, 0.0)


def _v_call(ss, vn, w1, b1, g1, be1, w2, b2, g2, be2):
  return pl.pallas_call(
      _v_body,
      out_shape=jax.ShapeDtypeStruct((G, D), jnp.float32),
  )(ss, vn, w1, b1, g1, be1, w2, b2, g2, be2)


def _pool_body(h_ref, b_ref, ss_ref):
  i = pl.program_id(0)
  ohT = (lax.broadcasted_iota(jnp.int32, (G, NB), 0) == b_ref[0]).astype(
      jnp.float32)
  part = lax.dot_general(ohT, h_ref[...], (((1,), (0,)), ((), ())),
                         preferred_element_type=jnp.float32)

  @pl.when(i == 0)
  def _():
    ss_ref[...] = jnp.zeros_like(ss_ref)
  ss_ref[...] += part


def _pool_call(h, batch3):
  return pl.pallas_call(
      _pool_body,
      grid=(NSTEP,),
      in_specs=[
          pl.BlockSpec((NB, D), lambda i: (i, 0)),
          pl.BlockSpec((1, 1, NB), lambda i: (i, 0, 0)),
      ],
      out_specs=pl.BlockSpec((G, D), lambda i: (0, 0)),
      out_shape=jax.ShapeDtypeStruct((G, D), jnp.float32),
  )(h, batch3)


def _pred_body(hg_ref, fp_ref, w1h_ref, w1f_ref, b1_ref, w2_ref, b2_ref,
               out_ref):
  z = (jnp.dot(hg_ref[...], w1h_ref[...], preferred_element_type=jnp.float32)
       + jnp.dot(fp_ref[...], w1f_ref[...], preferred_element_type=jnp.float32)
       + b1_ref[...])
  z = jax.nn.gelu(z)
  out_ref[...] = (
      jnp.dot(z, w2_ref[...], preferred_element_type=jnp.float32) + b2_ref[...])


def _pred_call(hg, fp, w1h, w1f, b1, w2, b2):
  return pl.pallas_call(
      _pred_body,
      out_shape=jax.ShapeDtypeStruct((G, NTASK), jnp.float32),
  )(hg, fp, w1h, w1f, b1, w2, b2)


def _etab_body(bf_ref, et_ref):
  ci = lax.broadcasted_iota(jnp.int32, (512, 24), 0)
  cj = lax.broadcasted_iota(jnp.int32, (512, 24), 1)
  oh = ((cj == (ci >> 6)) | (cj == 8 + ((ci >> 3) & 7)) |
        (cj == 16 + (ci & 7))).astype(jnp.float32)
  et_ref[0] = jnp.dot(oh, bf_ref[0], preferred_element_type=jnp.float32)


def _etab_call(bond_flat):
  return pl.pallas_call(
      _etab_body,
      grid=(NLAYER,),
      in_specs=[pl.BlockSpec((1, 24, D), lambda l: (l, 0, 0))],
      out_specs=pl.BlockSpec((1, 512, D), lambda l: (l, 0, 0)),
      out_shape=jax.ShapeDtypeStruct((NLAYER, 512, D), jnp.float32),
  )(bond_flat)


def kernel(x_feat, edge_index, edge_attr, batch, fp, atom_emb, vn_emb0,
           bond_emb, conv_eps, conv_W1, conv_b1, conv_g1, conv_be1, conv_W2,
           conv_b2, bn_g, bn_b, vn_W1, vn_b1, vn_g1, vn_be1, vn_W2, vn_b2,
           vn_g2, vn_be2, pred_W1, pred_b1, pred_W2, pred_b2):
  src = edge_index[0]
  dst = edge_index[1]

  c = dst // CH
  order = jnp.argsort(c)
  cs = c[order]
  counts = jnp.bincount(c, length=NCH)
  cnt_pad = ((counts + KE - 1) // KE) * KE
  starts_pad = jnp.concatenate(
      [jnp.zeros((1,), cnt_pad.dtype), jnp.cumsum(cnt_pad)[:-1]])
  starts_orig = jnp.cumsum(counts) - counts
  rank = jnp.arange(E, dtype=cs.dtype) - starts_orig[cs]
  pos = starts_pad[cs] + rank
  eidx = edge_attr[:, 0] * 64 + edge_attr[:, 1] * 8 + edge_attr[:, 2]
  src_p = jnp.zeros((TOT,), jnp.int32).at[pos].set(src[order])
  eidx_p = jnp.zeros((TOT,), jnp.int32).at[pos].set(eidx[order])
  ldst_p = jnp.full((TOT,), CH, jnp.int32).at[pos].set((dst - c * CH)[order])
  starts_w = jnp.zeros((NW, 16), jnp.int32).at[:, :RPW].set(
      starts_pad.reshape(NW, RPW).astype(jnp.int32))
  nit_w = jnp.zeros((NW, 16), jnp.int32).at[:, :RPW].set(
      (cnt_pad // KE).reshape(NW, RPW).astype(jnp.int32))

  atom_flat = _pad_d(atom_emb.reshape(576, EMB))
  bond_flat = _pad_d(bond_emb.reshape(NLAYER, 24, EMB))
  w1p = jnp.pad(conv_W1, ((0, 0), (0, D - EMB), (0, 0)))
  w2p = _pad_d(conv_W2)
  b2p = _pad_d(conv_b2)
  bgp = _pad_d(bn_g)
  bbp = _pad_d(bn_b)
  vw1p = jnp.pad(vn_W1, ((0, 0), (0, D - EMB), (0, 0)))
  vw2p = _pad_d(vn_W2)
  vb2p = _pad_d(vn_b2)
  vg2p = _pad_d(vn_g2)
  vbe2p = _pad_d(vn_be2)
  w1h = jnp.pad(pred_W1[:EMB], ((0, D - EMB), (0, 0)))
  w1f = pred_W1[EMB:]
  batch3 = batch.reshape(NSTEP, 1, NB)
  r1 = lambda x: x.reshape(1, -1)

  et = _etab_call(bond_flat)
  h = _atom_call(x_feat, atom_flat)
  vn = jnp.broadcast_to(_pad_d(vn_emb0), (G, D))

  for l in range(NLAYER):
    hin, ss = _a_call(h, batch3, vn)
    aggr = _sc_aggr(hin, et[l], src_p, eidx_p, ldst_p, starts_w, nit_w)[:N]
    h = _b_call(l < NLAYER - 1, hin, aggr, conv_eps[l].reshape(1, 1),
                w1p[l], r1(conv_b1[l]), r1(conv_g1[l]), r1(conv_be1[l]),
                w2p[l], r1(b2p[l]), r1(bgp[l]), r1(bbp[l]))
    if l < NLAYER - 1:
      vn = _v_call(ss, vn, vw1p[l], r1(vn_b1[l]), r1(vn_g1[l]),
                   r1(vn_be1[l]), vw2p[l], r1(vb2p[l]), r1(vg2p[l]),
                   r1(vbe2p[l]))

  hg = _pool_call(h, batch3)
  return _pred_call(hg, fp, w1h, w1f, r1(pred_b1), pred_W2, r1(pred_b2))


# speedup vs baseline: 4.6117x; 3.1865x over previous
import functools

import jax
import jax.numpy as jnp
from jax import lax
from jax.experimental import pallas as pl
from jax.experimental.pallas import tpu as pltpu
from jax.experimental.pallas import tpu_sc as plsc

EMB = 100
HID = 200
G = 256
NTASK = 128
NLAYER = 5
N = 50000
E = 800000
D = 128
CH = 200
NCH = 256
NPAD = CH * NCH
KE = 128
NW = 32
RPW = NCH // NW
NT = E // KE
NB = 2000
NSTEP = N // NB


def _pad_d(x):
  return jnp.pad(x, [(0, 0)] * (x.ndim - 1) + [(0, D - x.shape[-1])])


def _sc_body(hin, srcp, pkp, etab, startsw, out,
             etab_v, aggr_v, src_v, pk_v, hbuf, st_v, sem):
  w = lax.axis_index("s") * 2 + lax.axis_index("c")
  pltpu.sync_copy(etab, etab_v)
  pltpu.sync_copy(startsw.at[w], st_v)
  st_vec = st_v[...]
  zero16 = jnp.zeros((16,), jnp.float32)
  iota16 = lax.broadcasted_iota(jnp.int32, (16,), 0)
  ch_vec = jnp.full((16,), CH, jnp.int32)

  for cc in range(RPW):
    chunk = w * RPW + cc
    s = st_vec[cc]
    t = st_vec[cc + 1]
    tstart = s // KE
    tend = (t + KE - 1) // KE
    nt = tend - tstart

    def zrow(r, _):
      for j in range(D // 16):
        aggr_v[r, pl.ds(16 * j, 16)] = zero16
      return 0
    lax.fori_loop(0, CH + 1, zrow, 0)

    def etile(i, _):
      tt = tstart + i
      off = pl.multiple_of(tt * KE, KE)
      pltpu.sync_copy(srcp.at[pl.ds(off, KE)], src_v)
      pltpu.sync_copy(pkp.at[pl.ds(off, KE)], pk_v)
      pltpu.async_copy(hin.at[src_v], hbuf, sem).wait()

      def grp(g, _):
        pkvec = pk_v[pl.ds(g * 16, 16)]
        gvec = off + g * 16 + iota16
        mask = (gvec >= s) & (gvec < t)
        evec = pkvec >> 8
        lvec = jnp.where(mask, pkvec & 255, ch_vec)
        for k in range(16):
          li = lvec[k]
          ei = evec[k]
          row = g * 16 + k
          for j in range(D // 16):
            m = jnp.maximum(
                hbuf[row, pl.ds(16 * j, 16)]
                + etab_v[ei, pl.ds(16 * j, 16)], 0.0)
            plsc.addupdate(aggr_v.at[li, pl.ds(16 * j, 16)], m)
        return 0
      lax.fori_loop(0, KE // 16, grp, 0)
      return 0
    lax.fori_loop(0, nt, etile, 0)
    pltpu.sync_copy(aggr_v.at[pl.ds(0, CH)], out.at[pl.ds(chunk * CH, CH)])


_sc_aggr = functools.partial(
    pl.kernel,
    out_type=jax.ShapeDtypeStruct((NPAD, D), jnp.float32),
    mesh=plsc.VectorSubcoreMesh(core_axis_name="c", subcore_axis_name="s",
                                num_cores=2, num_subcores=16),
    scratch_types=[
        pltpu.VMEM((512, D), jnp.float32),
        pltpu.VMEM((CH + 1, D), jnp.float32),
        pltpu.VMEM((KE,), jnp.int32),
        pltpu.VMEM((KE,), jnp.int32),
        pltpu.VMEM((KE, D), jnp.float32),
        pltpu.VMEM((16,), jnp.int32),
        pltpu.SemaphoreType.DMA,
    ],
)(_sc_body)


def _atom_body(x_ref, af_ref, h_ref):
  x = x_ref[...]
  iota = lax.broadcasted_iota(jnp.int32, (NB, 576), 1)
  oh = jnp.zeros((NB, 576), jnp.float32)
  for i in range(9):
    oh += (iota == (x[:, i:i + 1] + 64 * i)).astype(jnp.float32)
  h_ref[...] = jnp.dot(oh, af_ref[...], preferred_element_type=jnp.float32)


def _atom_call(x_feat, atom_flat):
  return pl.pallas_call(
      _atom_body,
      grid=(NSTEP,),
      in_specs=[
          pl.BlockSpec((NB, 9), lambda i: (i, 0)),
          pl.BlockSpec((576, D), lambda i: (0, 0)),
      ],
      out_specs=pl.BlockSpec((NB, D), lambda i: (i, 0)),
      out_shape=jax.ShapeDtypeStruct((N, D), jnp.float32),
  )(x_feat, atom_flat)


def _a_body(h_ref, b_ref, vn_ref, hin_ref, ss_ref):
  i = pl.program_id(0)
  ohT = (lax.broadcasted_iota(jnp.int32, (G, NB), 0) == b_ref[0]).astype(
      jnp.float32)
  vnb = lax.dot_general(ohT, vn_ref[...], (((0,), (0,)), ((), ())),
                        preferred_element_type=jnp.float32)
  hin = h_ref[...] + vnb
  hin_ref[...] = hin
  part = lax.dot_general(ohT, hin, (((1,), (0,)), ((), ())),
                         preferred_element_type=jnp.float32)

  @pl.when(i == 0)
  def _():
    ss_ref[...] = jnp.zeros_like(ss_ref)
  ss_ref[...] += part


def _a_call(h, batch3, vn):
  return pl.pallas_call(
      _a_body,
      grid=(NSTEP,),
      in_specs=[
          pl.BlockSpec((NB, D), lambda i: (i, 0)),
          pl.BlockSpec((1, 1, NB), lambda i: (i, 0, 0)),
          pl.BlockSpec((G, D), lambda i: (0, 0)),
      ],
      out_specs=[
          pl.BlockSpec((NB, D), lambda i: (i, 0)),
          pl.BlockSpec((G, D), lambda i: (0, 0)),
      ],
      out_shape=[
          jax.ShapeDtypeStruct((N, D), jnp.float32),
          jax.ShapeDtypeStruct((G, D), jnp.float32),
      ],
  )(h, batch3, vn)


def _b_body(relu_out, hin_ref, ag_ref, eps_ref, w1_ref, b1_ref, g1_ref,
            be1_ref, w2_ref, b2_ref, bg_ref, bb_ref, out_ref):
  z = (1.0 + eps_ref[0, 0]) * hin_ref[...] + ag_ref[...]
  z = jnp.dot(z, w1_ref[...], preferred_element_type=jnp.float32) + b1_ref[...]
  z = z * g1_ref[...] + be1_ref[...]
  z = jnp.maximum(z, 0.0)
  z = jnp.dot(z, w2_ref[...], preferred_element_type=jnp.float32) + b2_ref[...]
  z = z * bg_ref[...] + bb_ref[...]
  if relu_out:
    z = jnp.maximum(z, 0.0)
  out_ref[...] = z


def _b_call(relu_out, hin, aggr, eps, w1, b1, g1, be1, w2, b2, bg, bb):
  row = lambda i: (0, 0)
  return pl.pallas_call(
      functools.partial(_b_body, relu_out),
      grid=(NSTEP,),
      in_specs=[
          pl.BlockSpec((NB, D), lambda i: (i, 0)),
          pl.BlockSpec((NB, D), lambda i: (i, 0)),
          pl.BlockSpec((1, 1), row),
          pl.BlockSpec((D, HID), row),
          pl.BlockSpec((1, HID), row),
          pl.BlockSpec((1, HID), row),
          pl.BlockSpec((1, HID), row),
          pl.BlockSpec((HID, D), row),
          pl.BlockSpec((1, D), row),
          pl.BlockSpec((1, D), row),
          pl.BlockSpec((1, D), row),
      ],
      out_specs=pl.BlockSpec((NB, D), lambda i: (i, 0)),
      out_shape=jax.ShapeDtypeStruct((N, D), jnp.float32),
  )(hin, aggr, eps, w1, b1, g1, be1, w2, b2, bg, bb)


def _v_body(ss_ref, vn_ref, w1_ref, b1_ref, g1_ref, be1_ref, w2_ref, b2_ref,
            g2_ref, be2_ref, out_ref):
  vt = ss_ref[...] + vn_ref[...]
  v = jnp.dot(vt, w1_ref[...], preferred_element_type=jnp.float32) + b1_ref[...]
  v = jnp.maximum(v * g1_ref[...] + be1_ref[...], 0.0)
  v = jnp.dot(v, w2_ref[...], preferred_element_type=jnp.float32) + b2_ref[...]
  v = v * g2_ref[...] + be2_ref[...]
  out_ref[...] = jnp.maximum(v, 0.0)


def _v_call(ss, vn, w1, b1, g1, be1, w2, b2, g2, be2):
  return pl.pallas_call(
      _v_body,
      out_shape=jax.ShapeDtypeStruct((G, D), jnp.float32),
  )(ss, vn, w1, b1, g1, be1, w2, b2, g2, be2)


def _pool_body(h_ref, b_ref, ss_ref):
  i = pl.program_id(0)
  ohT = (lax.broadcasted_iota(jnp.int32, (G, NB), 0) == b_ref[0]).astype(
      jnp.float32)
  part = lax.dot_general(ohT, h_ref[...], (((1,), (0,)), ((), ())),
                         preferred_element_type=jnp.float32)

  @pl.when(i == 0)
  def _():
    ss_ref[...] = jnp.zeros_like(ss_ref)
  ss_ref[...] += part


def _pool_call(h, batch3):
  return pl.pallas_call(
      _pool_body,
      grid=(NSTEP,),
      in_specs=[
          pl.BlockSpec((NB, D), lambda i: (i, 0)),
          pl.BlockSpec((1, 1, NB), lambda i: (i, 0, 0)),
      ],
      out_specs=pl.BlockSpec((G, D), lambda i: (0, 0)),
      out_shape=jax.ShapeDtypeStruct((G, D), jnp.float32),
  )(h, batch3)


def _pred_body(hg_ref, fp_ref, w1h_ref, w1f_ref, b1_ref, w2_ref, b2_ref,
               out_ref):
  z = (jnp.dot(hg_ref[...], w1h_ref[...], preferred_element_type=jnp.float32)
       + jnp.dot(fp_ref[...], w1f_ref[...], preferred_element_type=jnp.float32)
       + b1_ref[...])
  z = jax.nn.gelu(z)
  out_ref[...] = (
      jnp.dot(z, w2_ref[...], preferred_element_type=jnp.float32) + b2_ref[...])


def _pred_call(hg, fp, w1h, w1f, b1, w2, b2):
  return pl.pallas_call(
      _pred_body,
      out_shape=jax.ShapeDtypeStruct((G, NTASK), jnp.float32),
  )(hg, fp, w1h, w1f, b1, w2, b2)


def _etab_body(bf_ref, et_ref):
  ci = lax.broadcasted_iota(jnp.int32, (512, 24), 0)
  cj = lax.broadcasted_iota(jnp.int32, (512, 24), 1)
  oh = ((cj == (ci >> 6)) | (cj == 8 + ((ci >> 3) & 7)) |
        (cj == 16 + (ci & 7))).astype(jnp.float32)
  et_ref[0] = jnp.dot(oh, bf_ref[0], preferred_element_type=jnp.float32)


def _etab_call(bond_flat):
  return pl.pallas_call(
      _etab_body,
      grid=(NLAYER,),
      in_specs=[pl.BlockSpec((1, 24, D), lambda l: (l, 0, 0))],
      out_specs=pl.BlockSpec((1, 512, D), lambda l: (l, 0, 0)),
      out_shape=jax.ShapeDtypeStruct((NLAYER, 512, D), jnp.float32),
  )(bond_flat)


def kernel(x_feat, edge_index, edge_attr, batch, fp, atom_emb, vn_emb0,
           bond_emb, conv_eps, conv_W1, conv_b1, conv_g1, conv_be1, conv_W2,
           conv_b2, bn_g, bn_b, vn_W1, vn_b1, vn_g1, vn_be1, vn_W2, vn_b2,
           vn_g2, vn_be2, pred_W1, pred_b1, pred_W2, pred_b2):
  src = edge_index[0]
  dst = edge_index[1]

  c = dst // CH
  eidx = edge_attr[:, 0] * 64 + edge_attr[:, 1] * 8 + edge_attr[:, 2]
  pk = eidx * 256 + (dst - c * CH)
  cs, src_s, pk_s = lax.sort((c, src, pk), num_keys=1, is_stable=False)
  starts = jnp.searchsorted(cs, jnp.arange(NCH, dtype=cs.dtype)).astype(
      jnp.int32)
  starts_ext = jnp.concatenate(
      [starts, jnp.array([E], jnp.int32)])
  idx = (jnp.arange(NW)[:, None] * RPW
         + jnp.arange(RPW + 1)[None, :])
  starts_w = jnp.zeros((NW, 16), jnp.int32).at[:, :RPW + 1].set(
      starts_ext[idx])

  atom_flat = _pad_d(atom_emb.reshape(576, EMB))
  bond_flat = _pad_d(bond_emb.reshape(NLAYER, 24, EMB))
  w1p = jnp.pad(conv_W1, ((0, 0), (0, D - EMB), (0, 0)))
  w2p = _pad_d(conv_W2)
  b2p = _pad_d(conv_b2)
  bgp = _pad_d(bn_g)
  bbp = _pad_d(bn_b)
  vw1p = jnp.pad(vn_W1, ((0, 0), (0, D - EMB), (0, 0)))
  vw2p = _pad_d(vn_W2)
  vb2p = _pad_d(vn_b2)
  vg2p = _pad_d(vn_g2)
  vbe2p = _pad_d(vn_be2)
  w1h = jnp.pad(pred_W1[:EMB], ((0, D - EMB), (0, 0)))
  w1f = pred_W1[EMB:]
  batch3 = batch.reshape(NSTEP, 1, NB)
  r1 = lambda x: x.reshape(1, -1)

  et = _etab_call(bond_flat)
  h = _atom_call(x_feat, atom_flat)
  vn = jnp.broadcast_to(_pad_d(vn_emb0), (G, D))

  for l in range(NLAYER):
    hin, ss = _a_call(h, batch3, vn)
    if l < NLAYER - 1:
      vn = _v_call(ss, vn, vw1p[l], r1(vn_b1[l]), r1(vn_g1[l]),
                   r1(vn_be1[l]), vw2p[l], r1(vb2p[l]), r1(vg2p[l]),
                   r1(vbe2p[l]))
    aggr = _sc_aggr(hin, src_s, pk_s, et[l], starts_w)[:N]
    h = _b_call(l < NLAYER - 1, hin, aggr, conv_eps[l].reshape(1, 1),
                w1p[l], r1(conv_b1[l]), r1(conv_g1[l]), r1(conv_be1[l]),
                w2p[l], r1(b2p[l]), r1(bgp[l]), r1(bbp[l]))

  hg = _pool_call(h, batch3)
  return _pred_call(hg, fp, w1h, w1f, r1(pred_b1), pred_W2, r1(pred_b2))
